# Initial kernel scaffold; baseline (speedup 1.0000x reference)
import jax
import jax.numpy as jnp
from jax.experimental import pallas as pl


def kernel(edge_index, features, batch, W1, b1, W2, b2, att1_fc1, att1_fc2, att2_fc1, att2_fc2):
    raise NotImplementedError("write your pallas kernel here")


# speedup vs baseline: 13.0560x; 13.0560x over previous
import functools

import jax
import jax.numpy as jnp
from jax import lax
from jax.experimental import pallas as pl
from jax.experimental.pallas import tpu as pltpu
from jax.experimental.pallas import tpu_sc as plsc

_N = 10000
_E = 320000
_NG = 128

_NC = 2
_NS = 16
_NW = _NC * _NS
_K = 128
_CHUNKS = -(-_E // (_NW * _K))
_PER_W = _K * _CHUNKS
_EPAD = _NW * _PER_W
_NPAD = 10112
_RPT = _NPAD // _NS
_ROWBLK = 1000
_GRID = _N // _ROWBLK

_mesh = plsc.VectorSubcoreMesh(core_axis_name="c", subcore_axis_name="s")


_sc_params = pltpu.CompilerParams(use_tc_tiling_on_sc=False)


@functools.partial(
    pl.kernel, mesh=_mesh, compiler_params=_sc_params,
    out_type=jax.ShapeDtypeStruct((_NC, _NPAD, 16), jnp.float32),
    scratch_types=[
        pltpu.VMEM((_K,), jnp.int32),
        pltpu.VMEM((_K, 16), jnp.float32),
        pltpu.VMEM((_RPT, 16), jnp.float32),
        pltpu.VMEM_SHARED((_NPAD, 16), jnp.float32),
        pltpu.SemaphoreType.DMA,
    ])
def _sc_degree(dst_hbm, out_hbm, didx, ones_v, zbuf, acc, sem):
    c = lax.axis_index("c")
    s = lax.axis_index("s")
    wid = s * _NC + c

    def fill(i, _):
        ones_v[i, :] = jnp.ones((16,), jnp.float32)
        return 0
    lax.fori_loop(0, _K, fill, 0)

    def zfill(i, _):
        zbuf[i, :] = jnp.zeros((16,), jnp.float32)
        return 0
    lax.fori_loop(0, _RPT, zfill, 0)
    pltpu.sync_copy(zbuf, acc.at[pl.ds(s * _RPT, _RPT)])
    plsc.subcore_barrier()

    def body(i, _):
        base = wid * _PER_W + i * _K
        pltpu.sync_copy(dst_hbm.at[pl.ds(base, _K)], didx)
        pltpu.sync_copy(ones_v, acc.at[didx], add=True)
        return 0
    lax.fori_loop(0, _CHUNKS, body, 0)

    plsc.subcore_barrier()
    pltpu.sync_copy(acc.at[pl.ds(s * _RPT, _RPT)],
                    out_hbm.at[c, pl.ds(s * _RPT, _RPT)])


def _make_sc_scatter(width):
    @functools.partial(
        pl.kernel, mesh=_mesh, compiler_params=_sc_params,
        out_type=jax.ShapeDtypeStruct((_NC, _NPAD, width), jnp.float32),
        scratch_types=[
            pltpu.VMEM((_K,), jnp.int32),
            pltpu.VMEM((_K,), jnp.int32),
            pltpu.VMEM((_K, width), jnp.float32),
            pltpu.VMEM((_RPT, width), jnp.float32),
            pltpu.VMEM_SHARED((_NPAD, width), jnp.float32),
            pltpu.SemaphoreType.DMA,
        ])
    def k(tab_hbm, src_hbm, dst_hbm, out_hbm, sidx, didx, rows, zbuf, acc, sem):
        c = lax.axis_index("c")
        s = lax.axis_index("s")
        wid = s * _NC + c

        def zfill(i, _):
            def zcol(j, _2):
                zbuf[i, pl.ds(j * 16, 16)] = jnp.zeros((16,), jnp.float32)
                return 0
            return lax.fori_loop(0, width // 16, zcol, 0)
        lax.fori_loop(0, _RPT, zfill, 0)
        pltpu.sync_copy(zbuf, acc.at[pl.ds(s * _RPT, _RPT)])
        plsc.subcore_barrier()

        def body(i, _):
            base = wid * _PER_W + i * _K
            pltpu.sync_copy(src_hbm.at[pl.ds(base, _K)], sidx)
            pltpu.async_copy(tab_hbm.at[sidx], rows, sem).wait()
            pltpu.sync_copy(dst_hbm.at[pl.ds(base, _K)], didx)
            pltpu.sync_copy(rows, acc.at[didx], add=True)
            return 0
        lax.fori_loop(0, _CHUNKS, body, 0)

        plsc.subcore_barrier()
        pltpu.sync_copy(acc.at[pl.ds(s * _RPT, _RPT)],
                        out_hbm.at[c, pl.ds(s * _RPT, _RPT)])
    return k


_sc_scatter64 = _make_sc_scatter(64)
_sc_scatter32 = _make_sc_scatter(32)


def _row_spec(width):
    return pl.BlockSpec((_ROWBLK, width), lambda i: (i, 0))


def _full_spec(r, cols):
    return pl.BlockSpec((r, cols), lambda i: (0, 0))


def _tc_mm1(x, w):
    def body(x_ref, w_ref, o_ref):
        o_ref[...] = jnp.dot(x_ref[...], w_ref[...],
                             preferred_element_type=jnp.float32)
    return pl.pallas_call(
        body, grid=(_GRID,),
        in_specs=[_row_spec(128), _full_spec(128, 64)],
        out_specs=_row_spec(64),
        out_shape=jax.ShapeDtypeStruct((_N, 64), jnp.float32),
    )(x, w)


def _tc_scale1(d0, d1, h1):
    def body(d0_ref, d1_ref, h1_ref, g1_ref, dv_ref):
        deg = d0_ref[...][:, 0:1] + d1_ref[...][:, 0:1] + 1.0
        dv = lax.rsqrt(deg)
        g1_ref[...] = dv * h1_ref[...]
        dv_ref[...] = jnp.broadcast_to(dv, dv_ref.shape)
    return pl.pallas_call(
        body, grid=(_GRID,),
        in_specs=[_row_spec(16), _row_spec(16), _row_spec(64)],
        out_specs=[_row_spec(64), _row_spec(16)],
        out_shape=[jax.ShapeDtypeStruct((_N, 64), jnp.float32),
                   jax.ShapeDtypeStruct((_N, 16), jnp.float32)],
    )(d0, d1, h1)


def _tc_layer2(a0, a1, h1, dv, b1, w2):
    def body(a0_ref, a1_ref, h1_ref, dv_ref, b1_ref, w2_ref,
             f1_ref, h2_ref, g2_ref):
        dv = dv_ref[...][:, 0:1]
        f1 = jnp.maximum(dv * (a0_ref[...] + a1_ref[...])
                         + (dv * dv) * h1_ref[...] + b1_ref[0:1, :], 0.0)
        f1_ref[...] = f1
        h2 = jnp.dot(f1, w2_ref[...], preferred_element_type=jnp.float32)
        h2_ref[...] = h2
        g2_ref[...] = dv * h2
    return pl.pallas_call(
        body, grid=(_GRID,),
        in_specs=[_row_spec(64), _row_spec(64), _row_spec(64), _row_spec(16),
                  _full_spec(8, 64), _full_spec(64, 32)],
        out_specs=[_row_spec(64), _row_spec(32), _row_spec(32)],
        out_shape=[jax.ShapeDtypeStruct((_N, 64), jnp.float32),
                   jax.ShapeDtypeStruct((_N, 32), jnp.float32),
                   jax.ShapeDtypeStruct((_N, 32), jnp.float32)],
    )(a0, a1, h1, dv, b1, w2)


def _tc_pool1(a0, a1, h2, dv, b2, f1, bt, w11, w12, w21, w22):
    def body(a0_ref, a1_ref, h2_ref, dv_ref, b2_ref, f1_ref, bt_ref,
             w11_ref, w12_ref, w21_ref, w22_ref,
             x1_ref, x2_ref, s1_ref, s2_ref, cnt_ref):
        i = pl.program_id(0)
        dv = dv_ref[...][:, 0:1]
        f2 = jnp.maximum(dv * (a0_ref[...] + a1_ref[...])
                         + (dv * dv) * h2_ref[...] + b2_ref[0:1, :], 0.0)
        f1 = f1_ref[...]
        att1 = jnp.tanh(jnp.dot(
            jnp.maximum(jnp.dot(f1, w11_ref[...],
                                preferred_element_type=jnp.float32), 0.0),
            w12_ref[...], preferred_element_type=jnp.float32))
        x1 = f1 + att1 * f1
        att2 = jnp.tanh(jnp.dot(
            jnp.maximum(jnp.dot(f2, w21_ref[...],
                                preferred_element_type=jnp.float32), 0.0),
            w22_ref[...], preferred_element_type=jnp.float32))
        x2 = f2 + att2 * f2
        x1_ref[...] = x1
        x2_ref[...] = x2
        b = bt_ref[...][:, 0:1]
        oh = (b == lax.broadcasted_iota(jnp.int32, (_ROWBLK, _NG), 1)
              ).astype(jnp.float32)
        seg1 = lax.dot_general(oh, x1, (((0,), (0,)), ((), ())),
                               preferred_element_type=jnp.float32)
        seg2 = lax.dot_general(oh, x2, (((0,), (0,)), ((), ())),
                               preferred_element_type=jnp.float32)
        cnt = jnp.sum(oh, axis=0)[:, None]

        @pl.when(i == 0)
        def _():
            s1_ref[...] = jnp.zeros_like(s1_ref)
            s2_ref[...] = jnp.zeros_like(s2_ref)
            cnt_ref[...] = jnp.zeros_like(cnt_ref)
        s1_ref[...] += seg1
        s2_ref[...] += seg2
        cnt_ref[...] += jnp.broadcast_to(cnt, cnt_ref.shape)
    return pl.pallas_call(
        body, grid=(_GRID,),
        in_specs=[_row_spec(32), _row_spec(32), _row_spec(32), _row_spec(16),
                  _full_spec(8, 32), _row_spec(64), _row_spec(16),
                  _full_spec(64, 16), _full_spec(16, 64),
                  _full_spec(32, 8), _full_spec(8, 32)],
        out_specs=[_row_spec(64), _row_spec(32),
                   _full_spec(_NG, 64), _full_spec(_NG, 32),
                   _full_spec(_NG, 16)],
        out_shape=[jax.ShapeDtypeStruct((_N, 64), jnp.float32),
                   jax.ShapeDtypeStruct((_N, 32), jnp.float32),
                   jax.ShapeDtypeStruct((_NG, 64), jnp.float32),
                   jax.ShapeDtypeStruct((_NG, 32), jnp.float32),
                   jax.ShapeDtypeStruct((_NG, 16), jnp.float32)],
    )(a0, a1, h2, dv, b2, f1, bt, w11, w12, w21, w22)


def _tc_pool2(x1p, x2p, bt, s1, s2, cnt):
    def body(x1_ref, x2_ref, bt_ref, s1_ref, s2_ref, cnt_ref, o1_ref, o2_ref):
        i = pl.program_id(0)
        cntm = jnp.maximum(cnt_ref[...][:, 0:1], 1.0)
        tg1 = jnp.tanh(s1_ref[...] / cntm)
        tg2 = jnp.tanh(s2_ref[...] / cntm)
        b = bt_ref[...][:, 0:1]
        oh = (b == lax.broadcasted_iota(jnp.int32, (_ROWBLK, _NG), 1)
              ).astype(jnp.float32)
        t1 = jnp.dot(oh, tg1, preferred_element_type=jnp.float32)
        t2 = jnp.dot(oh, tg2, preferred_element_type=jnp.float32)
        x1 = x1_ref[...]
        x2 = x2_ref[...]
        c1 = jax.nn.sigmoid(jnp.sum(x1 * t1, axis=1, keepdims=True))
        c2 = jax.nn.sigmoid(jnp.sum(x2 * t2, axis=1, keepdims=True))
        p1 = lax.dot_general(oh, c1 * x1, (((0,), (0,)), ((), ())),
                             preferred_element_type=jnp.float32)
        p2 = lax.dot_general(oh, c2 * x2, (((0,), (0,)), ((), ())),
                             preferred_element_type=jnp.float32)

        @pl.when(i == 0)
        def _():
            o1_ref[...] = jnp.zeros_like(o1_ref)
            o2_ref[...] = jnp.zeros_like(o2_ref)
        o1_ref[...] += p1
        o2_ref[...] += p2
    return pl.pallas_call(
        body, grid=(_GRID,),
        in_specs=[_row_spec(64), _row_spec(32), _row_spec(16),
                  _full_spec(_NG, 64), _full_spec(_NG, 32),
                  _full_spec(_NG, 16)],
        out_specs=[_full_spec(_NG, 64), _full_spec(_NG, 32)],
        out_shape=[jax.ShapeDtypeStruct((_NG, 64), jnp.float32),
                   jax.ShapeDtypeStruct((_NG, 32), jnp.float32)],
    )(x1p, x2p, bt, s1, s2, cnt)


def _b2d(b):
    return jnp.broadcast_to(b[None, :], (8, b.shape[0]))


def kernel(edge_index, features, batch, W1, b1, W2, b2,
           att1_fc1, att1_fc2, att2_fc1, att2_fc2):
    src, dst = edge_index[0], edge_index[1]
    pad = _EPAD - _E
    src_p = jnp.concatenate([src, jnp.zeros((pad,), jnp.int32)])
    dst_p = jnp.concatenate([dst, jnp.full((pad,), _N, jnp.int32)])

    d_parts = _sc_degree(dst_p)
    h1 = _tc_mm1(features, W1)
    g1, dv = _tc_scale1(d_parts[0, :_N, :], d_parts[1, :_N, :], h1)
    acc1 = _sc_scatter64(g1, src_p, dst_p)
    f1, h2, g2 = _tc_layer2(acc1[0, :_N, :], acc1[1, :_N, :], h1, dv,
                            _b2d(b1), W2)
    acc2 = _sc_scatter32(g2, src_p, dst_p)
    bt = jnp.broadcast_to(batch[:, None], (_N, 16))
    x1p, x2p, s1, s2, cnt = _tc_pool1(acc2[0, :_N, :], acc2[1, :_N, :], h2,
                                      dv, _b2d(b2), f1, bt,
                                      att1_fc1, att1_fc2, att2_fc1, att2_fc2)
    p1, p2 = _tc_pool2(x1p, x2p, bt, s1, s2, cnt)
    return jnp.concatenate((p2, p1), axis=1)


# speedup vs baseline: 22.1710x; 1.6981x over previous
import functools

import jax
import jax.numpy as jnp
from jax import lax
from jax.experimental import pallas as pl
from jax.experimental.pallas import tpu as pltpu
from jax.experimental.pallas import tpu_sc as plsc

_N = 10000
_E = 320000
_NG = 128

_NC = 2
_NS = 16
_NW = _NC * _NS
_K = 128
_CHUNKS = -(-_E // (_NW * _K))
_PER_W = _K * _CHUNKS
_EPAD = _NW * _PER_W
_NPAD = 10112
_RPT = _NPAD // _NS
_ROWBLK = 1000
_GRID = _N // _ROWBLK

_mesh = plsc.VectorSubcoreMesh(core_axis_name="c", subcore_axis_name="s")


_sc_params = pltpu.CompilerParams(use_tc_tiling_on_sc=False)
_NB = 4


@functools.partial(
    pl.kernel, mesh=_mesh, compiler_params=_sc_params,
    out_type=jax.ShapeDtypeStruct((_NC, _NPAD, 16), jnp.float32),
    scratch_types=[
        pltpu.VMEM((_CHUNKS, _K), jnp.int32),
        pltpu.VMEM((_K, 16), jnp.float32),
        pltpu.VMEM((_RPT, 16), jnp.float32),
        pltpu.VMEM_SHARED((_NPAD, 16), jnp.float32),
        pltpu.SemaphoreType.DMA,
    ])
def _sc_degree(dst_hbm, out_hbm, didx, ones_v, zbuf, acc, sem):
    c = lax.axis_index("c")
    s = lax.axis_index("s")
    wid = s * _NC + c

    def fill(i, _):
        ones_v[i, :] = jnp.ones((16,), jnp.float32)
        return 0
    lax.fori_loop(0, _K, fill, 0)

    def zfill(i, _):
        zbuf[i, :] = jnp.zeros((16,), jnp.float32)
        return 0
    lax.fori_loop(0, _RPT, zfill, 0)
    pltpu.sync_copy(zbuf, acc.at[pl.ds(s * _RPT, _RPT)])
    pltpu.sync_copy(dst_hbm.at[wid], didx)
    plsc.subcore_barrier()

    descs = [pltpu.make_async_copy(ones_v, acc.at[didx.at[i]], sem)
             for i in range(_CHUNKS)]
    for d in descs:
        d.start(add=True)
    for d in descs:
        d.wait()

    plsc.subcore_barrier()
    pltpu.sync_copy(acc.at[pl.ds(s * _RPT, _RPT)],
                    out_hbm.at[c, pl.ds(s * _RPT, _RPT)])


def _make_sc_scatter(width):
    @functools.partial(
        pl.kernel, mesh=_mesh, compiler_params=_sc_params,
        out_type=jax.ShapeDtypeStruct((_NC, _NPAD, width), jnp.float32),
        scratch_types=[
            pltpu.VMEM((_CHUNKS, _K), jnp.int32),
            pltpu.VMEM((_CHUNKS, _K), jnp.int32),
            pltpu.VMEM((_NB, _K, width), jnp.float32),
            pltpu.VMEM_SHARED((_NPAD, width), jnp.float32),
            pltpu.SemaphoreType.DMA((_NB,)),
            pltpu.SemaphoreType.DMA((_NB,)),
        ])
    def k(tab_hbm, src_hbm, dst_hbm, out_hbm, sidx, didx, rows, acc,
          gsem, ssem):
        c = lax.axis_index("c")
        s = lax.axis_index("s")
        wid = s * _NC + c

        def zfill(i, _):
            def zcol(j, _2):
                rows[0, i, pl.ds(j * 16, 16)] = jnp.zeros((16,), jnp.float32)
                return 0
            return lax.fori_loop(0, width // 16, zcol, 0)
        lax.fori_loop(0, _K, zfill, 0)
        nfull, rem = divmod(_RPT, _K)
        for t in range(nfull):
            pltpu.sync_copy(rows.at[0],
                            acc.at[pl.ds(s * _RPT + t * _K, _K)])
        if rem:
            pltpu.sync_copy(rows.at[0].at[pl.ds(0, rem)],
                            acc.at[pl.ds(s * _RPT + nfull * _K, rem)])
        pltpu.sync_copy(src_hbm.at[wid], sidx)
        pltpu.sync_copy(dst_hbm.at[wid], didx)
        plsc.subcore_barrier()

        def g_desc(j):
            return pltpu.make_async_copy(
                tab_hbm.at[sidx.at[j]], rows.at[j % _NB], gsem.at[j % _NB])

        def s_desc(i):
            return pltpu.make_async_copy(
                rows.at[i % _NB], acc.at[didx.at[i]], ssem.at[i % _NB])

        g_desc(0).start()
        g_desc(1).start()
        for i in range(_CHUNKS):
            j = i + 2
            if j < _CHUNKS:
                if j >= _NB:
                    s_desc(j - _NB).wait()
                g_desc(j).start()
            g_desc(i).wait()
            s_desc(i).start(add=True)
        for i in range(max(0, _CHUNKS - _NB), _CHUNKS):
            s_desc(i).wait()

        plsc.subcore_barrier()
        pltpu.sync_copy(acc.at[pl.ds(s * _RPT, _RPT)],
                        out_hbm.at[c, pl.ds(s * _RPT, _RPT)])
    return k


_sc_scatter64 = _make_sc_scatter(64)
_sc_scatter32 = _make_sc_scatter(32)


def _row_spec(width):
    return pl.BlockSpec((_ROWBLK, width), lambda i: (i, 0))


def _full_spec(r, cols):
    return pl.BlockSpec((r, cols), lambda i: (0, 0))


def _tc_mm1(x, w):
    def body(x_ref, w_ref, o_ref):
        o_ref[...] = jnp.dot(x_ref[...], w_ref[...],
                             preferred_element_type=jnp.float32)
    return pl.pallas_call(
        body, grid=(_GRID,),
        in_specs=[_row_spec(128), _full_spec(128, 64)],
        out_specs=_row_spec(64),
        out_shape=jax.ShapeDtypeStruct((_N, 64), jnp.float32),
    )(x, w)


def _tc_scale1(d0, d1, h1):
    def body(d0_ref, d1_ref, h1_ref, g1_ref, dv_ref):
        deg = d0_ref[...][:, 0:1] + d1_ref[...][:, 0:1] + 1.0
        dv = lax.rsqrt(deg)
        g1_ref[...] = dv * h1_ref[...]
        dv_ref[...] = jnp.broadcast_to(dv, dv_ref.shape)
    return pl.pallas_call(
        body, grid=(_GRID,),
        in_specs=[_row_spec(16), _row_spec(16), _row_spec(64)],
        out_specs=[_row_spec(64), _row_spec(16)],
        out_shape=[jax.ShapeDtypeStruct((_N, 64), jnp.float32),
                   jax.ShapeDtypeStruct((_N, 16), jnp.float32)],
    )(d0, d1, h1)


def _tc_layer2(a0, a1, h1, dv, b1, w2):
    def body(a0_ref, a1_ref, h1_ref, dv_ref, b1_ref, w2_ref,
             f1_ref, h2_ref, g2_ref):
        dv = dv_ref[...][:, 0:1]
        f1 = jnp.maximum(dv * (a0_ref[...] + a1_ref[...])
                         + (dv * dv) * h1_ref[...] + b1_ref[0:1, :], 0.0)
        f1_ref[...] = f1
        h2 = jnp.dot(f1, w2_ref[...], preferred_element_type=jnp.float32)
        h2_ref[...] = h2
        g2_ref[...] = dv * h2
    return pl.pallas_call(
        body, grid=(_GRID,),
        in_specs=[_row_spec(64), _row_spec(64), _row_spec(64), _row_spec(16),
                  _full_spec(8, 64), _full_spec(64, 32)],
        out_specs=[_row_spec(64), _row_spec(32), _row_spec(32)],
        out_shape=[jax.ShapeDtypeStruct((_N, 64), jnp.float32),
                   jax.ShapeDtypeStruct((_N, 32), jnp.float32),
                   jax.ShapeDtypeStruct((_N, 32), jnp.float32)],
    )(a0, a1, h1, dv, b1, w2)


def _tc_pool1(a0, a1, h2, dv, b2, f1, bt, w11, w12, w21, w22):
    def body(a0_ref, a1_ref, h2_ref, dv_ref, b2_ref, f1_ref, bt_ref,
             w11_ref, w12_ref, w21_ref, w22_ref,
             x1_ref, x2_ref, s1_ref, s2_ref, cnt_ref):
        i = pl.program_id(0)
        dv = dv_ref[...][:, 0:1]
        f2 = jnp.maximum(dv * (a0_ref[...] + a1_ref[...])
                         + (dv * dv) * h2_ref[...] + b2_ref[0:1, :], 0.0)
        f1 = f1_ref[...]
        att1 = jnp.tanh(jnp.dot(
            jnp.maximum(jnp.dot(f1, w11_ref[...],
                                preferred_element_type=jnp.float32), 0.0),
            w12_ref[...], preferred_element_type=jnp.float32))
        x1 = f1 + att1 * f1
        att2 = jnp.tanh(jnp.dot(
            jnp.maximum(jnp.dot(f2, w21_ref[...],
                                preferred_element_type=jnp.float32), 0.0),
            w22_ref[...], preferred_element_type=jnp.float32))
        x2 = f2 + att2 * f2
        x1_ref[...] = x1
        x2_ref[...] = x2
        b = bt_ref[...][:, 0:1]
        oh = (b == lax.broadcasted_iota(jnp.int32, (_ROWBLK, _NG), 1)
              ).astype(jnp.float32)
        seg1 = lax.dot_general(oh, x1, (((0,), (0,)), ((), ())),
                               preferred_element_type=jnp.float32)
        seg2 = lax.dot_general(oh, x2, (((0,), (0,)), ((), ())),
                               preferred_element_type=jnp.float32)
        cnt = jnp.sum(oh, axis=0)[:, None]

        @pl.when(i == 0)
        def _():
            s1_ref[...] = jnp.zeros_like(s1_ref)
            s2_ref[...] = jnp.zeros_like(s2_ref)
            cnt_ref[...] = jnp.zeros_like(cnt_ref)
        s1_ref[...] += seg1
        s2_ref[...] += seg2
        cnt_ref[...] += jnp.broadcast_to(cnt, cnt_ref.shape)
    return pl.pallas_call(
        body, grid=(_GRID,),
        in_specs=[_row_spec(32), _row_spec(32), _row_spec(32), _row_spec(16),
                  _full_spec(8, 32), _row_spec(64), _row_spec(16),
                  _full_spec(64, 16), _full_spec(16, 64),
                  _full_spec(32, 8), _full_spec(8, 32)],
        out_specs=[_row_spec(64), _row_spec(32),
                   _full_spec(_NG, 64), _full_spec(_NG, 32),
                   _full_spec(_NG, 16)],
        out_shape=[jax.ShapeDtypeStruct((_N, 64), jnp.float32),
                   jax.ShapeDtypeStruct((_N, 32), jnp.float32),
                   jax.ShapeDtypeStruct((_NG, 64), jnp.float32),
                   jax.ShapeDtypeStruct((_NG, 32), jnp.float32),
                   jax.ShapeDtypeStruct((_NG, 16), jnp.float32)],
    )(a0, a1, h2, dv, b2, f1, bt, w11, w12, w21, w22)


def _tc_pool2(x1p, x2p, bt, s1, s2, cnt):
    def body(x1_ref, x2_ref, bt_ref, s1_ref, s2_ref, cnt_ref, o1_ref, o2_ref):
        i = pl.program_id(0)
        cntm = jnp.maximum(cnt_ref[...][:, 0:1], 1.0)
        tg1 = jnp.tanh(s1_ref[...] / cntm)
        tg2 = jnp.tanh(s2_ref[...] / cntm)
        b = bt_ref[...][:, 0:1]
        oh = (b == lax.broadcasted_iota(jnp.int32, (_ROWBLK, _NG), 1)
              ).astype(jnp.float32)
        t1 = jnp.dot(oh, tg1, preferred_element_type=jnp.float32)
        t2 = jnp.dot(oh, tg2, preferred_element_type=jnp.float32)
        x1 = x1_ref[...]
        x2 = x2_ref[...]
        c1 = jax.nn.sigmoid(jnp.sum(x1 * t1, axis=1, keepdims=True))
        c2 = jax.nn.sigmoid(jnp.sum(x2 * t2, axis=1, keepdims=True))
        p1 = lax.dot_general(oh, c1 * x1, (((0,), (0,)), ((), ())),
                             preferred_element_type=jnp.float32)
        p2 = lax.dot_general(oh, c2 * x2, (((0,), (0,)), ((), ())),
                             preferred_element_type=jnp.float32)

        @pl.when(i == 0)
        def _():
            o1_ref[...] = jnp.zeros_like(o1_ref)
            o2_ref[...] = jnp.zeros_like(o2_ref)
        o1_ref[...] += p1
        o2_ref[...] += p2
    return pl.pallas_call(
        body, grid=(_GRID,),
        in_specs=[_row_spec(64), _row_spec(32), _row_spec(16),
                  _full_spec(_NG, 64), _full_spec(_NG, 32),
                  _full_spec(_NG, 16)],
        out_specs=[_full_spec(_NG, 64), _full_spec(_NG, 32)],
        out_shape=[jax.ShapeDtypeStruct((_NG, 64), jnp.float32),
                   jax.ShapeDtypeStruct((_NG, 32), jnp.float32)],
    )(x1p, x2p, bt, s1, s2, cnt)


def _b2d(b):
    return jnp.broadcast_to(b[None, :], (8, b.shape[0]))


def kernel(edge_index, features, batch, W1, b1, W2, b2,
           att1_fc1, att1_fc2, att2_fc1, att2_fc2):
    src, dst = edge_index[0], edge_index[1]
    pad = _EPAD - _E
    src_p = jnp.concatenate([src, jnp.zeros((pad,), jnp.int32)]
                            ).reshape(_NW, _CHUNKS, _K)
    dst_p = jnp.concatenate([dst, jnp.full((pad,), _N, jnp.int32)]
                            ).reshape(_NW, _CHUNKS, _K)

    d_parts = _sc_degree(dst_p)
    h1 = _tc_mm1(features, W1)
    g1, dv = _tc_scale1(d_parts[0, :_N, :], d_parts[1, :_N, :], h1)
    acc1 = _sc_scatter64(g1, src_p, dst_p)
    f1, h2, g2 = _tc_layer2(acc1[0, :_N, :], acc1[1, :_N, :], h1, dv,
                            _b2d(b1), W2)
    acc2 = _sc_scatter32(g2, src_p, dst_p)
    bt = jnp.broadcast_to(batch[:, None], (_N, 16))
    x1p, x2p, s1, s2, cnt = _tc_pool1(acc2[0, :_N, :], acc2[1, :_N, :], h2,
                                      dv, _b2d(b2), f1, bt,
                                      att1_fc1, att1_fc2, att2_fc1, att2_fc2)
    p1, p2 = _tc_pool2(x1p, x2p, bt, s1, s2, cnt)
    return jnp.concatenate((p2, p1), axis=1)


# speedup vs baseline: 23.2450x; 1.0484x over previous
import functools

import jax
import jax.numpy as jnp
from jax import lax
from jax.experimental import pallas as pl
from jax.experimental.pallas import tpu as pltpu
from jax.experimental.pallas import tpu_sc as plsc

_N = 10000
_E = 320000
_NG = 128

_NC = 2
_NS = 16
_NW = _NC * _NS
_K = 128
_CHUNKS = -(-_E // (_NW * _K))
_PER_W = _K * _CHUNKS
_EPAD = _NW * _PER_W
_NPAD = 10112
_RPT = _NPAD // _NS
_ROWBLK = 1000
_GRID = _N // _ROWBLK

_mesh = plsc.VectorSubcoreMesh(core_axis_name="c", subcore_axis_name="s")


_sc_params = pltpu.CompilerParams(use_tc_tiling_on_sc=False)
_NB = 4


@functools.partial(
    pl.kernel, mesh=_mesh, compiler_params=_sc_params,
    out_type=jax.ShapeDtypeStruct((_NC, _NPAD, 16), jnp.float32),
    scratch_types=[
        pltpu.VMEM((_CHUNKS, _K), jnp.int32),
        pltpu.VMEM((_K, 16), jnp.float32),
        pltpu.VMEM((_RPT, 16), jnp.float32),
        pltpu.VMEM_SHARED((_NPAD, 16), jnp.float32),
        pltpu.SemaphoreType.DMA,
    ])
def _sc_degree(dst_hbm, out_hbm, didx, ones_v, zbuf, acc, sem):
    c = lax.axis_index("c")
    s = lax.axis_index("s")
    wid = s * _NC + c

    def fill(i, _):
        ones_v[i, :] = jnp.ones((16,), jnp.float32)
        return 0
    lax.fori_loop(0, _K, fill, 0)

    def zfill(i, _):
        zbuf[i, :] = jnp.zeros((16,), jnp.float32)
        return 0
    lax.fori_loop(0, _RPT, zfill, 0)
    pltpu.sync_copy(zbuf, acc.at[pl.ds(s * _RPT, _RPT)])
    pltpu.sync_copy(dst_hbm.at[wid], didx)
    plsc.subcore_barrier()

    descs = [pltpu.make_async_copy(ones_v, acc.at[didx.at[i]], sem)
             for i in range(_CHUNKS)]
    for d in descs:
        d.start(add=True)
    for d in descs:
        d.wait()

    plsc.subcore_barrier()
    pltpu.sync_copy(acc.at[pl.ds(s * _RPT, _RPT)],
                    out_hbm.at[c, pl.ds(s * _RPT, _RPT)])


def _make_sc_scatter(width):
    @functools.partial(
        pl.kernel, mesh=_mesh, compiler_params=_sc_params,
        out_type=jax.ShapeDtypeStruct((_NC, _NPAD, width), jnp.float32),
        scratch_types=[
            pltpu.VMEM((_CHUNKS, _K), jnp.int32),
            pltpu.VMEM((_CHUNKS, _K), jnp.int32),
            pltpu.VMEM((_NB, _K, width), jnp.float32),
            pltpu.VMEM_SHARED((_NPAD, width), jnp.float32),
            pltpu.SemaphoreType.DMA((_NB,)),
            pltpu.SemaphoreType.DMA((_NB,)),
        ])
    def k(tab_hbm, src_hbm, dst_hbm, out_hbm, sidx, didx, rows, acc,
          gsem, ssem):
        c = lax.axis_index("c")
        s = lax.axis_index("s")
        wid = s * _NC + c

        def zfill(i, _):
            def zcol(j, _2):
                rows[0, i, pl.ds(j * 16, 16)] = jnp.zeros((16,), jnp.float32)
                return 0
            return lax.fori_loop(0, width // 16, zcol, 0)
        lax.fori_loop(0, _K, zfill, 0)
        nfull, rem = divmod(_RPT, _K)
        for t in range(nfull):
            pltpu.sync_copy(rows.at[0],
                            acc.at[pl.ds(s * _RPT + t * _K, _K)])
        if rem:
            pltpu.sync_copy(rows.at[0].at[pl.ds(0, rem)],
                            acc.at[pl.ds(s * _RPT + nfull * _K, rem)])
        pltpu.sync_copy(src_hbm.at[wid], sidx)
        pltpu.sync_copy(dst_hbm.at[wid], didx)
        plsc.subcore_barrier()

        def g_desc(j):
            return pltpu.make_async_copy(
                tab_hbm.at[sidx.at[j]], rows.at[j % _NB], gsem.at[j % _NB])

        def s_desc(i):
            return pltpu.make_async_copy(
                rows.at[i % _NB], acc.at[didx.at[i]], ssem.at[i % _NB])

        g_desc(0).start()
        g_desc(1).start()
        for i in range(_CHUNKS):
            j = i + 2
            if j < _CHUNKS:
                if j >= _NB:
                    s_desc(j - _NB).wait()
                g_desc(j).start()
            g_desc(i).wait()
            s_desc(i).start(add=True)
        for i in range(max(0, _CHUNKS - _NB), _CHUNKS):
            s_desc(i).wait()

        plsc.subcore_barrier()
        pltpu.sync_copy(acc.at[pl.ds(s * _RPT, _RPT)],
                        out_hbm.at[c, pl.ds(s * _RPT, _RPT)])
    return k


_sc_scatter64 = _make_sc_scatter(64)
_sc_scatter32 = _make_sc_scatter(32)


def _row_spec(width):
    return pl.BlockSpec((_ROWBLK, width), lambda i: (i, 0))


def _part_spec(core, width):
    return pl.BlockSpec((1, _ROWBLK, width), lambda i: (core, i, 0))


def _full_spec(r, cols):
    return pl.BlockSpec((r, cols), lambda i: (0, 0))


def _tc_mm1(x, w):
    def body(x_ref, w_ref, o_ref):
        o_ref[...] = jnp.dot(x_ref[...], w_ref[...],
                             preferred_element_type=jnp.float32)
    return pl.pallas_call(
        body, grid=(_GRID,),
        in_specs=[_row_spec(128), _full_spec(128, 64)],
        out_specs=_row_spec(64),
        out_shape=jax.ShapeDtypeStruct((_N, 64), jnp.float32),
    )(x, w)


def _tc_scale1(d_parts, h1):
    def body(d0_ref, d1_ref, h1_ref, g1_ref, dv_ref):
        deg = d0_ref[0][:, 0:1] + d1_ref[0][:, 0:1] + 1.0
        dv = lax.rsqrt(deg)
        g1_ref[...] = dv * h1_ref[...]
        dv_ref[...] = jnp.broadcast_to(dv, dv_ref.shape)
    return pl.pallas_call(
        body, grid=(_GRID,),
        in_specs=[_part_spec(0, 16), _part_spec(1, 16), _row_spec(64)],
        out_specs=[_row_spec(64), _row_spec(16)],
        out_shape=[jax.ShapeDtypeStruct((_N, 64), jnp.float32),
                   jax.ShapeDtypeStruct((_N, 16), jnp.float32)],
    )(d_parts, d_parts, h1)


def _tc_layer2(acc1, h1, dv, b1, w2, w11, w12, bt):
    def body(a0_ref, a1_ref, h1_ref, dv_ref, b1_ref, w2_ref,
             w11_ref, w12_ref, bt_ref,
             h2_ref, g2_ref, x1_ref, s1_ref, cnt_ref):
        i = pl.program_id(0)
        dv = dv_ref[...][:, 0:1]
        f1 = jnp.maximum(dv * (a0_ref[0] + a1_ref[0])
                         + (dv * dv) * h1_ref[...] + b1_ref[0:1, :], 0.0)
        h2 = jnp.dot(f1, w2_ref[...], preferred_element_type=jnp.float32)
        h2_ref[...] = h2
        g2_ref[...] = dv * h2
        att1 = jnp.tanh(jnp.dot(
            jnp.maximum(jnp.dot(f1, w11_ref[...],
                                preferred_element_type=jnp.float32), 0.0),
            w12_ref[...], preferred_element_type=jnp.float32))
        x1 = f1 + att1 * f1
        x1_ref[...] = x1
        b = bt_ref[...][:, 0:1]
        oh = (b == lax.broadcasted_iota(jnp.int32, (_ROWBLK, _NG), 1)
              ).astype(jnp.float32)
        seg1 = lax.dot_general(oh, x1, (((0,), (0,)), ((), ())),
                               preferred_element_type=jnp.float32)
        cnt = jnp.sum(oh, axis=0)[:, None]

        @pl.when(i == 0)
        def _():
            s1_ref[...] = jnp.zeros_like(s1_ref)
            cnt_ref[...] = jnp.zeros_like(cnt_ref)
        s1_ref[...] += seg1
        cnt_ref[...] += jnp.broadcast_to(cnt, cnt_ref.shape)
    return pl.pallas_call(
        body, grid=(_GRID,),
        in_specs=[_part_spec(0, 64), _part_spec(1, 64), _row_spec(64),
                  _row_spec(16), _full_spec(8, 64), _full_spec(64, 32),
                  _full_spec(64, 16), _full_spec(16, 64), _row_spec(16)],
        out_specs=[_row_spec(32), _row_spec(32), _row_spec(64),
                   _full_spec(_NG, 64), _full_spec(_NG, 16)],
        out_shape=[jax.ShapeDtypeStruct((_N, 32), jnp.float32),
                   jax.ShapeDtypeStruct((_N, 32), jnp.float32),
                   jax.ShapeDtypeStruct((_N, 64), jnp.float32),
                   jax.ShapeDtypeStruct((_NG, 64), jnp.float32),
                   jax.ShapeDtypeStruct((_NG, 16), jnp.float32)],
    )(acc1, acc1, h1, dv, b1, w2, w11, w12, bt)


def _tc_pool(acc2, h2, dv, b2, bt, x1, s1, cnt, w21, w22):
    def body(a0_ref, a1_ref, h2_ref, dv_ref, b2_ref, bt_ref, x1_ref,
             s1_ref, cnt_ref, w21_ref, w22_ref, o_ref, x2v, s2v):
        i = pl.program_id(0)

        @pl.when(i == 0)
        def _():
            s2v[...] = jnp.zeros_like(s2v)
            o_ref[...] = jnp.zeros_like(o_ref)

        b = bt_ref[...][:, 0:1]
        oh = (b == lax.broadcasted_iota(jnp.int32, (_ROWBLK, _NG), 1)
              ).astype(jnp.float32)

        @pl.when(i < _GRID)
        def _():
            dv = dv_ref[...][:, 0:1]
            f2 = jnp.maximum(dv * (a0_ref[0] + a1_ref[0])
                             + (dv * dv) * h2_ref[...] + b2_ref[0:1, :], 0.0)
            att2 = jnp.tanh(jnp.dot(
                jnp.maximum(jnp.dot(f2, w21_ref[...],
                                    preferred_element_type=jnp.float32), 0.0),
                w22_ref[...], preferred_element_type=jnp.float32))
            x2 = f2 + att2 * f2
            x2v[pl.ds(i * _ROWBLK, _ROWBLK), :] = x2
            s2v[...] += lax.dot_general(oh, x2, (((0,), (0,)), ((), ())),
                                        preferred_element_type=jnp.float32)

        @pl.when(i >= _GRID)
        def _():
            ib = i - _GRID
            cntm = jnp.maximum(cnt_ref[...][:, 0:1], 1.0)
            tg1 = jnp.tanh(s1_ref[...] / cntm)
            tg2 = jnp.tanh(s2v[...] / cntm)
            x1 = x1_ref[...]
            x2 = x2v[pl.ds(ib * _ROWBLK, _ROWBLK), :]
            t1 = jnp.dot(oh, tg1, preferred_element_type=jnp.float32)
            t2 = jnp.dot(oh, tg2, preferred_element_type=jnp.float32)
            c1 = jax.nn.sigmoid(jnp.sum(x1 * t1, axis=1, keepdims=True))
            c2 = jax.nn.sigmoid(jnp.sum(x2 * t2, axis=1, keepdims=True))
            p2 = lax.dot_general(oh, c2 * x2, (((0,), (0,)), ((), ())),
                                 preferred_element_type=jnp.float32)
            p1 = lax.dot_general(oh, c1 * x1, (((0,), (0,)), ((), ())),
                                 preferred_element_type=jnp.float32)
            o_ref[:, 0:32] += p2
            o_ref[:, 32:96] += p1

    def _p1_idx(i):
        return jnp.where(i < _GRID, i, _GRID - 1)

    def _p2_idx(i):
        return jnp.where(i < _GRID, 0, i - _GRID)

    def _both_idx(i):
        return jnp.where(i < _GRID, i, i - _GRID)

    return pl.pallas_call(
        body, grid=(2 * _GRID,),
        in_specs=[
            pl.BlockSpec((1, _ROWBLK, 32), lambda i: (0, _p1_idx(i), 0)),
            pl.BlockSpec((1, _ROWBLK, 32), lambda i: (1, _p1_idx(i), 0)),
            pl.BlockSpec((_ROWBLK, 32), lambda i: (_p1_idx(i), 0)),
            pl.BlockSpec((_ROWBLK, 16), lambda i: (_p1_idx(i), 0)),
            pl.BlockSpec((8, 32), lambda i: (0, 0)),
            pl.BlockSpec((_ROWBLK, 16), lambda i: (_both_idx(i), 0)),
            pl.BlockSpec((_ROWBLK, 64), lambda i: (_p2_idx(i), 0)),
            pl.BlockSpec((_NG, 64), lambda i: (0, 0)),
            pl.BlockSpec((_NG, 16), lambda i: (0, 0)),
            pl.BlockSpec((32, 8), lambda i: (0, 0)),
            pl.BlockSpec((8, 32), lambda i: (0, 0)),
        ],
        out_specs=pl.BlockSpec((_NG, 96), lambda i: (0, 0)),
        out_shape=jax.ShapeDtypeStruct((_NG, 96), jnp.float32),
        scratch_shapes=[pltpu.VMEM((_N, 32), jnp.float32),
                        pltpu.VMEM((_NG, 32), jnp.float32)],
    )(acc2, acc2, h2, dv, b2, bt, x1, s1, cnt, w21, w22)


def _b2d(b):
    return jnp.broadcast_to(b[None, :], (8, b.shape[0]))


def kernel(edge_index, features, batch, W1, b1, W2, b2,
           att1_fc1, att1_fc2, att2_fc1, att2_fc2):
    src, dst = edge_index[0], edge_index[1]
    pad = _EPAD - _E
    src_p = jnp.concatenate([src, jnp.zeros((pad,), jnp.int32)]
                            ).reshape(_NW, _CHUNKS, _K)
    dst_p = jnp.concatenate([dst, jnp.full((pad,), _N, jnp.int32)]
                            ).reshape(_NW, _CHUNKS, _K)
    bt = jnp.broadcast_to(batch[:, None], (_N, 16))

    d_parts = _sc_degree(dst_p)
    h1 = _tc_mm1(features, W1)
    g1, dv = _tc_scale1(d_parts, h1)
    acc1 = _sc_scatter64(g1, src_p, dst_p)
    h2, g2, x1, s1, cnt = _tc_layer2(acc1, h1, dv, _b2d(b1), W2,
                                     att1_fc1, att1_fc2, bt)
    acc2 = _sc_scatter32(g2, src_p, dst_p)
    return _tc_pool(acc2, h2, dv, _b2d(b2), bt, x1, s1, cnt,
                    att2_fc1, att2_fc2)


# speedup vs baseline: 34.8884x; 1.5009x over previous
import functools

import jax
import jax.numpy as jnp
from jax import lax
from jax.experimental import pallas as pl
from jax.experimental.pallas import tpu as pltpu
from jax.experimental.pallas import tpu_sc as plsc

_N = 10000
_E = 320000
_NG = 128

_NC = 2
_NS = 16
_NW = _NC * _NS
_K = 128
_CHUNKS = -(-_E // (_NW * _K))
_PER_W = _K * _CHUNKS
_EPAD = _NW * _PER_W
_NPAD = 10112
_RPT = _NPAD // _NS
_ROWBLK = 1000
_GRID = _N // _ROWBLK

_mesh = plsc.VectorSubcoreMesh(core_axis_name="c", subcore_axis_name="s")


_sc_params = pltpu.CompilerParams(use_tc_tiling_on_sc=False)
_NB = 4


@functools.partial(
    pl.kernel, mesh=_mesh, compiler_params=_sc_params,
    out_type=jax.ShapeDtypeStruct((_NC, _NPAD, 16), jnp.float32),
    scratch_types=[
        pltpu.VMEM((_CHUNKS, _K), jnp.int32),
        pltpu.VMEM((_K, 16), jnp.float32),
        pltpu.VMEM((_RPT, 16), jnp.float32),
        pltpu.VMEM_SHARED((_NPAD, 16), jnp.float32),
        pltpu.SemaphoreType.DMA,
    ])
def _sc_degree(dst_hbm, out_hbm, didx, ones_v, zbuf, acc, sem):
    c = lax.axis_index("c")
    s = lax.axis_index("s")
    wid = s * _NC + c

    def fill(i, _):
        ones_v[i, :] = jnp.ones((16,), jnp.float32)
        return 0
    lax.fori_loop(0, _K, fill, 0)

    def zfill(i, _):
        zbuf[i, :] = jnp.zeros((16,), jnp.float32)
        return 0
    lax.fori_loop(0, _RPT, zfill, 0)
    pltpu.sync_copy(zbuf, acc.at[pl.ds(s * _RPT, _RPT)])
    pltpu.sync_copy(dst_hbm.at[wid], didx)
    plsc.subcore_barrier()

    descs = [pltpu.make_async_copy(ones_v, acc.at[didx.at[i]], sem)
             for i in range(_CHUNKS)]
    for d in descs:
        d.start(add=True)
    for d in descs:
        d.wait()

    plsc.subcore_barrier()
    pltpu.sync_copy(acc.at[pl.ds(s * _RPT, _RPT)],
                    out_hbm.at[c, pl.ds(s * _RPT, _RPT)])


def _make_sc_scatter(width):
    @functools.partial(
        pl.kernel, mesh=_mesh, compiler_params=_sc_params,
        out_type=jax.ShapeDtypeStruct((_NC, _NPAD, width), jnp.float32),
        scratch_types=[
            pltpu.VMEM((_CHUNKS, _K), jnp.int32),
            pltpu.VMEM((_CHUNKS, _K), jnp.int32),
            pltpu.VMEM((_NB, _K, width), jnp.float32),
            pltpu.VMEM_SHARED((_NPAD, width), jnp.float32),
            pltpu.SemaphoreType.DMA((_NB,)),
            pltpu.SemaphoreType.DMA((_NB,)),
        ])
    def k(tab_hbm, src_hbm, dst_hbm, out_hbm, sidx, didx, rows, acc,
          gsem, ssem):
        c = lax.axis_index("c")
        s = lax.axis_index("s")
        wid = s * _NC + c

        def zfill(i, _):
            def zcol(j, _2):
                rows[0, i, pl.ds(j * 16, 16)] = jnp.zeros((16,), jnp.float32)
                return 0
            return lax.fori_loop(0, width // 16, zcol, 0)
        lax.fori_loop(0, _K, zfill, 0)
        nfull, rem = divmod(_RPT, _K)
        for t in range(nfull):
            pltpu.sync_copy(rows.at[0],
                            acc.at[pl.ds(s * _RPT + t * _K, _K)])
        if rem:
            pltpu.sync_copy(rows.at[0].at[pl.ds(0, rem)],
                            acc.at[pl.ds(s * _RPT + nfull * _K, rem)])
        pltpu.sync_copy(src_hbm.at[wid], sidx)
        pltpu.sync_copy(dst_hbm.at[wid], didx)
        plsc.subcore_barrier()

        def g_desc(j):
            return pltpu.make_async_copy(
                tab_hbm.at[sidx.at[j]], rows.at[j % _NB], gsem.at[j % _NB])

        def s_desc(i):
            return pltpu.make_async_copy(
                rows.at[i % _NB], acc.at[didx.at[i]], ssem.at[i % _NB])

        g_desc(0).start()
        g_desc(1).start()
        for i in range(_CHUNKS):
            j = i + 2
            if j < _CHUNKS:
                if j >= _NB:
                    s_desc(j - _NB).wait()
                g_desc(j).start()
            g_desc(i).wait()
            s_desc(i).start(add=True)
        for i in range(max(0, _CHUNKS - _NB), _CHUNKS):
            s_desc(i).wait()

        plsc.subcore_barrier()
        pltpu.sync_copy(acc.at[pl.ds(s * _RPT, _RPT)],
                        out_hbm.at[c, pl.ds(s * _RPT, _RPT)])
    return k


_sc_scatter64 = _make_sc_scatter(64)
_sc_scatter32 = _make_sc_scatter(32)


def _row_spec(width):
    return pl.BlockSpec((_ROWBLK, width), lambda i: (i, 0))


def _part_spec(core, width):
    return pl.BlockSpec((1, _ROWBLK, width), lambda i: (core, i, 0))


def _full_spec(r, cols):
    return pl.BlockSpec((r, cols), lambda i: (0, 0))


def _tc_mm1(x, w):
    def body(x_ref, w_ref, o_ref):
        o_ref[...] = jnp.dot(x_ref[...], w_ref[...],
                             preferred_element_type=jnp.float32)
    return pl.pallas_call(
        body, grid=(_GRID,),
        in_specs=[_row_spec(128), _full_spec(128, 64)],
        out_specs=_row_spec(64),
        out_shape=jax.ShapeDtypeStruct((_N, 64), jnp.float32),
    )(x, w)


def _tc_scale1(d_parts, h1):
    def body(d0_ref, d1_ref, h1_ref, g1_ref, dv_ref):
        deg = d0_ref[0][:, 0:1] + d1_ref[0][:, 0:1] + 1.0
        dv = lax.rsqrt(deg)
        g1_ref[...] = dv * h1_ref[...]
        dv_ref[...] = jnp.broadcast_to(dv, dv_ref.shape)
    return pl.pallas_call(
        body, grid=(_GRID,),
        in_specs=[_part_spec(0, 16), _part_spec(1, 16), _row_spec(64)],
        out_specs=[_row_spec(64), _row_spec(16)],
        out_shape=[jax.ShapeDtypeStruct((_N, 64), jnp.float32),
                   jax.ShapeDtypeStruct((_N, 16), jnp.float32)],
    )(d_parts, d_parts, h1)


def _tc_layer2(acc1, h1, dv, b1, w2, w11, w12, bt):
    def body(a0_ref, a1_ref, h1_ref, dv_ref, b1_ref, w2_ref,
             w11_ref, w12_ref, bt_ref,
             h2_ref, g2_ref, x1_ref, s1_ref, cnt_ref):
        i = pl.program_id(0)
        dv = dv_ref[...][:, 0:1]
        f1 = jnp.maximum(dv * (a0_ref[0] + a1_ref[0])
                         + (dv * dv) * h1_ref[...] + b1_ref[0:1, :], 0.0)
        h2 = jnp.dot(f1, w2_ref[...], preferred_element_type=jnp.float32)
        h2_ref[...] = h2
        g2_ref[...] = dv * h2
        att1 = jnp.tanh(jnp.dot(
            jnp.maximum(jnp.dot(f1, w11_ref[...],
                                preferred_element_type=jnp.float32), 0.0),
            w12_ref[...], preferred_element_type=jnp.float32))
        x1 = f1 + att1 * f1
        x1_ref[...] = x1
        b = bt_ref[...][:, 0:1]
        oh = (b == lax.broadcasted_iota(jnp.int32, (_ROWBLK, _NG), 1)
              ).astype(jnp.float32)
        seg1 = lax.dot_general(oh, x1, (((0,), (0,)), ((), ())),
                               preferred_element_type=jnp.float32)
        cnt = jnp.sum(oh, axis=0)[:, None]

        @pl.when(i == 0)
        def _():
            s1_ref[...] = jnp.zeros_like(s1_ref)
            cnt_ref[...] = jnp.zeros_like(cnt_ref)
        s1_ref[...] += seg1
        cnt_ref[...] += jnp.broadcast_to(cnt, cnt_ref.shape)
    return pl.pallas_call(
        body, grid=(_GRID,),
        in_specs=[_part_spec(0, 64), _part_spec(1, 64), _row_spec(64),
                  _row_spec(16), _full_spec(8, 64), _full_spec(64, 32),
                  _full_spec(64, 16), _full_spec(16, 64), _row_spec(16)],
        out_specs=[_row_spec(32), _row_spec(32), _row_spec(64),
                   _full_spec(_NG, 64), _full_spec(_NG, 16)],
        out_shape=[jax.ShapeDtypeStruct((_N, 32), jnp.float32),
                   jax.ShapeDtypeStruct((_N, 32), jnp.float32),
                   jax.ShapeDtypeStruct((_N, 64), jnp.float32),
                   jax.ShapeDtypeStruct((_NG, 64), jnp.float32),
                   jax.ShapeDtypeStruct((_NG, 16), jnp.float32)],
    )(acc1, acc1, h1, dv, b1, w2, w11, w12, bt)


def _tc_pool(acc2, h2, dv, b2, bt, x1, s1, cnt, w21, w22):
    def body(a0_ref, a1_ref, h2_ref, dv_ref, b2_ref, bt_ref, x1_ref,
             s1_ref, cnt_ref, w21_ref, w22_ref, o_ref, x2v, s2v):
        i = pl.program_id(0)

        @pl.when(i == 0)
        def _():
            s2v[...] = jnp.zeros_like(s2v)
            o_ref[...] = jnp.zeros_like(o_ref)

        b = bt_ref[...][:, 0:1]
        oh = (b == lax.broadcasted_iota(jnp.int32, (_ROWBLK, _NG), 1)
              ).astype(jnp.float32)

        @pl.when(i < _GRID)
        def _():
            dv = dv_ref[...][:, 0:1]
            f2 = jnp.maximum(dv * (a0_ref[0] + a1_ref[0])
                             + (dv * dv) * h2_ref[...] + b2_ref[0:1, :], 0.0)
            att2 = jnp.tanh(jnp.dot(
                jnp.maximum(jnp.dot(f2, w21_ref[...],
                                    preferred_element_type=jnp.float32), 0.0),
                w22_ref[...], preferred_element_type=jnp.float32))
            x2 = f2 + att2 * f2
            x2v[pl.ds(i * _ROWBLK, _ROWBLK), :] = x2
            s2v[...] += lax.dot_general(oh, x2, (((0,), (0,)), ((), ())),
                                        preferred_element_type=jnp.float32)

        @pl.when(i >= _GRID)
        def _():
            ib = i - _GRID
            cntm = jnp.maximum(cnt_ref[...][:, 0:1], 1.0)
            tg1 = jnp.tanh(s1_ref[...] / cntm)
            tg2 = jnp.tanh(s2v[...] / cntm)
            x1 = x1_ref[...]
            x2 = x2v[pl.ds(ib * _ROWBLK, _ROWBLK), :]
            t1 = jnp.dot(oh, tg1, preferred_element_type=jnp.float32)
            t2 = jnp.dot(oh, tg2, preferred_element_type=jnp.float32)
            c1 = jax.nn.sigmoid(jnp.sum(x1 * t1, axis=1, keepdims=True))
            c2 = jax.nn.sigmoid(jnp.sum(x2 * t2, axis=1, keepdims=True))
            p2 = lax.dot_general(oh, c2 * x2, (((0,), (0,)), ((), ())),
                                 preferred_element_type=jnp.float32)
            p1 = lax.dot_general(oh, c1 * x1, (((0,), (0,)), ((), ())),
                                 preferred_element_type=jnp.float32)
            o_ref[:, 0:32] += p2
            o_ref[:, 32:96] += p1

    def _p1_idx(i):
        return jnp.where(i < _GRID, i, _GRID - 1)

    def _p2_idx(i):
        return jnp.where(i < _GRID, 0, i - _GRID)

    def _both_idx(i):
        return jnp.where(i < _GRID, i, i - _GRID)

    return pl.pallas_call(
        body, grid=(2 * _GRID,),
        in_specs=[
            pl.BlockSpec((1, _ROWBLK, 32), lambda i: (0, _p1_idx(i), 0)),
            pl.BlockSpec((1, _ROWBLK, 32), lambda i: (1, _p1_idx(i), 0)),
            pl.BlockSpec((_ROWBLK, 32), lambda i: (_p1_idx(i), 0)),
            pl.BlockSpec((_ROWBLK, 16), lambda i: (_p1_idx(i), 0)),
            pl.BlockSpec((8, 32), lambda i: (0, 0)),
            pl.BlockSpec((_ROWBLK, 16), lambda i: (_both_idx(i), 0)),
            pl.BlockSpec((_ROWBLK, 64), lambda i: (_p2_idx(i), 0)),
            pl.BlockSpec((_NG, 64), lambda i: (0, 0)),
            pl.BlockSpec((_NG, 16), lambda i: (0, 0)),
            pl.BlockSpec((32, 8), lambda i: (0, 0)),
            pl.BlockSpec((8, 32), lambda i: (0, 0)),
        ],
        out_specs=pl.BlockSpec((_NG, 96), lambda i: (0, 0)),
        out_shape=jax.ShapeDtypeStruct((_NG, 96), jnp.float32),
        scratch_shapes=[pltpu.VMEM((_N, 32), jnp.float32),
                        pltpu.VMEM((_NG, 32), jnp.float32)],
    )(acc2, acc2, h2, dv, b2, bt, x1, s1, cnt, w21, w22)


def _b2d(b):
    return jnp.broadcast_to(b[None, :], (8, b.shape[0]))


def kernel(edge_index, features, batch, W1, b1, W2, b2,
           att1_fc1, att1_fc2, att2_fc1, att2_fc2):
    src, dst = edge_index[0], edge_index[1]
    pad = _EPAD - _E
    spread = jnp.arange(pad, dtype=jnp.int32)
    src_p = jnp.concatenate([src, spread % _N]).reshape(_NW, _CHUNKS, _K)
    dst_p = jnp.concatenate([dst, _N + spread % (_NPAD - _N)]
                            ).reshape(_NW, _CHUNKS, _K)
    bt = jnp.broadcast_to(batch[:, None], (_N, 16))

    d_parts = _sc_degree(dst_p)
    h1 = _tc_mm1(features, W1)
    g1, dv = _tc_scale1(d_parts, h1)
    acc1 = _sc_scatter64(g1, src_p, dst_p)
    h2, g2, x1, s1, cnt = _tc_layer2(acc1, h1, dv, _b2d(b1), W2,
                                     att1_fc1, att1_fc2, bt)
    acc2 = _sc_scatter32(g2, src_p, dst_p)
    return _tc_pool(acc2, h2, dv, _b2d(b2), bt, x1, s1, cnt,
                    att2_fc1, att2_fc2)


# speedup vs baseline: 35.1827x; 1.0084x over previous
import functools

import jax
import jax.numpy as jnp
from jax import lax
from jax.experimental import pallas as pl
from jax.experimental.pallas import tpu as pltpu
from jax.experimental.pallas import tpu_sc as plsc

_N = 10000
_E = 320000
_NG = 128

_NC = 2
_NS = 16
_NW = _NC * _NS
_K = 128
_CHUNKS = -(-_E // (_NW * _K))
_PER_W = _K * _CHUNKS
_EPAD = _NW * _PER_W
_NPAD = 10112
_RPT = _NPAD // _NS
_ROWBLK = 1000
_GRID = _N // _ROWBLK

_mesh = plsc.VectorSubcoreMesh(core_axis_name="c", subcore_axis_name="s")


_sc_params = pltpu.CompilerParams(use_tc_tiling_on_sc=False)
_NB = 4


@functools.partial(
    pl.kernel, mesh=_mesh, compiler_params=_sc_params,
    out_type=jax.ShapeDtypeStruct((_NC, _NPAD, 16), jnp.float32),
    scratch_types=[
        pltpu.VMEM((_CHUNKS, _K), jnp.int32),
        pltpu.VMEM((_K, 16), jnp.float32),
        pltpu.VMEM((_RPT, 16), jnp.float32),
        pltpu.VMEM_SHARED((_NPAD, 16), jnp.float32),
        pltpu.SemaphoreType.DMA,
    ])
def _sc_degree(dst_hbm, out_hbm, didx, ones_v, zbuf, acc, sem):
    c = lax.axis_index("c")
    s = lax.axis_index("s")
    wid = s * _NC + c

    def fill(i, _):
        ones_v[i, :] = jnp.ones((16,), jnp.float32)
        return 0
    lax.fori_loop(0, _K, fill, 0)

    def zfill(i, _):
        zbuf[i, :] = jnp.zeros((16,), jnp.float32)
        return 0
    lax.fori_loop(0, _RPT, zfill, 0)
    pltpu.sync_copy(zbuf, acc.at[pl.ds(s * _RPT, _RPT)])
    pltpu.sync_copy(dst_hbm.at[wid], didx)
    plsc.subcore_barrier()

    descs = [pltpu.make_async_copy(ones_v, acc.at[didx.at[i]], sem)
             for i in range(_CHUNKS)]
    for d in descs:
        d.start(add=True)
    for d in descs:
        d.wait()

    plsc.subcore_barrier()
    pltpu.sync_copy(acc.at[pl.ds(s * _RPT, _RPT)],
                    out_hbm.at[c, pl.ds(s * _RPT, _RPT)])


def _make_sc_scatter(width):
    @functools.partial(
        pl.kernel, mesh=_mesh, compiler_params=_sc_params,
        out_type=jax.ShapeDtypeStruct((_NC, _NPAD, width), jnp.float32),
        scratch_types=[
            pltpu.VMEM((_CHUNKS, _K), jnp.int32),
            pltpu.VMEM((_CHUNKS, _K), jnp.int32),
            pltpu.VMEM((_NB, _K, width), jnp.float32),
            pltpu.VMEM_SHARED((_NPAD, width), jnp.float32),
            pltpu.SemaphoreType.DMA((_NB,)),
            pltpu.SemaphoreType.DMA((_NB,)),
        ])
    def k(tab_hbm, src_hbm, dst_hbm, out_hbm, sidx, didx, rows, acc,
          gsem, ssem):
        c = lax.axis_index("c")
        s = lax.axis_index("s")
        wid = s * _NC + c

        def zfill(i, _):
            def zcol(j, _2):
                rows[0, i, pl.ds(j * 16, 16)] = jnp.zeros((16,), jnp.float32)
                return 0
            return lax.fori_loop(0, width // 16, zcol, 0)
        lax.fori_loop(0, _K, zfill, 0)
        nfull, rem = divmod(_RPT, _K)
        for t in range(nfull):
            pltpu.sync_copy(rows.at[0],
                            acc.at[pl.ds(s * _RPT + t * _K, _K)])
        if rem:
            pltpu.sync_copy(rows.at[0].at[pl.ds(0, rem)],
                            acc.at[pl.ds(s * _RPT + nfull * _K, rem)])
        pltpu.sync_copy(src_hbm.at[wid], sidx)
        pltpu.sync_copy(dst_hbm.at[wid], didx)
        plsc.subcore_barrier()

        def g_desc(j):
            return pltpu.make_async_copy(
                tab_hbm.at[sidx.at[j]], rows.at[j % _NB], gsem.at[j % _NB])

        def s_desc(i):
            return pltpu.make_async_copy(
                rows.at[i % _NB], acc.at[didx.at[i]], ssem.at[i % _NB])

        g_desc(0).start()
        g_desc(1).start()
        for i in range(_CHUNKS):
            j = i + 2
            if j < _CHUNKS:
                if j >= _NB:
                    s_desc(j - _NB).wait()
                g_desc(j).start()
            g_desc(i).wait()
            s_desc(i).start(add=True)
        for i in range(max(0, _CHUNKS - _NB), _CHUNKS):
            s_desc(i).wait()

        plsc.subcore_barrier()
        pltpu.sync_copy(acc.at[pl.ds(s * _RPT, _RPT)],
                        out_hbm.at[c, pl.ds(s * _RPT, _RPT)])
    return k


_sc_scatter64 = _make_sc_scatter(64)
_sc_scatter32 = _make_sc_scatter(32)


def _row_spec(width):
    return pl.BlockSpec((_ROWBLK, width), lambda i: (i, 0))


def _part_spec(core, width):
    return pl.BlockSpec((1, _ROWBLK, width), lambda i: (core, i, 0))


def _full_spec(r, cols):
    return pl.BlockSpec((r, cols), lambda i: (0, 0))


def _tc_mm1(x, w):
    def body(x_ref, w_ref, o_ref):
        o_ref[...] = jnp.dot(x_ref[...], w_ref[...],
                             preferred_element_type=jnp.float32)
    return pl.pallas_call(
        body, grid=(_GRID,),
        in_specs=[_row_spec(128), _full_spec(128, 64)],
        out_specs=_row_spec(64),
        out_shape=jax.ShapeDtypeStruct((_N, 64), jnp.float32),
    )(x, w)


def _tc_scale1(d_parts, h1):
    def body(d0_ref, d1_ref, h1_ref, g1_ref, dv_ref):
        deg = d0_ref[0][:, 0:1] + d1_ref[0][:, 0:1] + 1.0
        dv = lax.rsqrt(deg)
        g1_ref[...] = dv * h1_ref[...]
        dv_ref[...] = jnp.broadcast_to(dv, dv_ref.shape)
    return pl.pallas_call(
        body, grid=(_GRID,),
        in_specs=[_part_spec(0, 16), _part_spec(1, 16), _row_spec(64)],
        out_specs=[_row_spec(64), _row_spec(16)],
        out_shape=[jax.ShapeDtypeStruct((_N, 64), jnp.float32),
                   jax.ShapeDtypeStruct((_N, 16), jnp.float32)],
    )(d_parts, d_parts, h1)


def _tc_layer2(acc1, g1, dv, b1, w2, w11, w12, bt):
    def body(a0_ref, a1_ref, g1_ref, dv_ref, b1_ref, w2_ref,
             w11_ref, w12_ref, bt_ref,
             g2_ref, x1_ref, s1_ref, cnt_ref):
        i = pl.program_id(0)
        dv = dv_ref[...][:, 0:1]
        f1 = jnp.maximum(dv * (a0_ref[0] + a1_ref[0] + g1_ref[...])
                         + b1_ref[0:1, :], 0.0)
        h2 = jnp.dot(f1, w2_ref[...], preferred_element_type=jnp.float32)
        g2_ref[...] = dv * h2
        att1 = jnp.tanh(jnp.dot(
            jnp.maximum(jnp.dot(f1, w11_ref[...],
                                preferred_element_type=jnp.float32), 0.0),
            w12_ref[...], preferred_element_type=jnp.float32))
        x1 = f1 + att1 * f1
        x1_ref[...] = x1
        bk = bt_ref[...][:, 0:1]
        oht = (lax.broadcasted_iota(jnp.int32, (_NG, _ROWBLK), 0)
               == bk.reshape(1, _ROWBLK)).astype(jnp.float32)
        seg1 = jnp.dot(oht, x1, preferred_element_type=jnp.float32)
        cnt = jnp.sum(oht, axis=1)[:, None]

        @pl.when(i == 0)
        def _():
            s1_ref[...] = jnp.zeros_like(s1_ref)
            cnt_ref[...] = jnp.zeros_like(cnt_ref)
        s1_ref[...] += seg1
        cnt_ref[...] += jnp.broadcast_to(cnt, cnt_ref.shape)
    return pl.pallas_call(
        body, grid=(_GRID,),
        in_specs=[_part_spec(0, 64), _part_spec(1, 64), _row_spec(64),
                  _row_spec(16), _full_spec(8, 64), _full_spec(64, 32),
                  _full_spec(64, 16), _full_spec(16, 64), _row_spec(16)],
        out_specs=[_row_spec(32), _row_spec(64),
                   _full_spec(_NG, 64), _full_spec(_NG, 16)],
        out_shape=[jax.ShapeDtypeStruct((_N, 32), jnp.float32),
                   jax.ShapeDtypeStruct((_N, 64), jnp.float32),
                   jax.ShapeDtypeStruct((_NG, 64), jnp.float32),
                   jax.ShapeDtypeStruct((_NG, 16), jnp.float32)],
    )(acc1, acc1, g1, dv, b1, w2, w11, w12, bt)


def _tc_pool(acc2, g2, dv, b2, bt, x1, s1, cnt, w21, w22):
    def body(a0_ref, a1_ref, g2_ref, dv_ref, b2_ref, bt_ref, x1_ref,
             s1_ref, cnt_ref, w21_ref, w22_ref, o_ref, x2v, s2v):
        i = pl.program_id(0)

        @pl.when(i == 0)
        def _():
            s2v[...] = jnp.zeros_like(s2v)
            o_ref[...] = jnp.zeros_like(o_ref)

        bk = bt_ref[...][:, 0:1]
        oht = (lax.broadcasted_iota(jnp.int32, (_NG, _ROWBLK), 0)
               == bk.reshape(1, _ROWBLK)).astype(jnp.float32)

        @pl.when(i < _GRID)
        def _():
            dv = dv_ref[...][:, 0:1]
            f2 = jnp.maximum(dv * (a0_ref[0] + a1_ref[0] + g2_ref[...])
                             + b2_ref[0:1, :], 0.0)
            att2 = jnp.tanh(jnp.dot(
                jnp.maximum(jnp.dot(f2, w21_ref[...],
                                    preferred_element_type=jnp.float32), 0.0),
                w22_ref[...], preferred_element_type=jnp.float32))
            x2 = f2 + att2 * f2
            x2v[pl.ds(i * _ROWBLK, _ROWBLK), :] = x2
            s2v[...] += jnp.dot(oht, x2, preferred_element_type=jnp.float32)

        @pl.when(i >= _GRID)
        def _():
            ib = i - _GRID
            oh = (bk == lax.broadcasted_iota(jnp.int32, (_ROWBLK, _NG), 1)
                  ).astype(jnp.float32)
            cntm = jnp.maximum(cnt_ref[...][:, 0:1], 1.0)
            tg1 = jnp.tanh(s1_ref[...] / cntm)
            tg2 = jnp.tanh(s2v[...] / cntm)
            x1 = x1_ref[...]
            x2 = x2v[pl.ds(ib * _ROWBLK, _ROWBLK), :]
            t1 = jnp.dot(oh, tg1, preferred_element_type=jnp.float32)
            t2 = jnp.dot(oh, tg2, preferred_element_type=jnp.float32)
            c1 = jax.nn.sigmoid(jnp.sum(x1 * t1, axis=1, keepdims=True))
            c2 = jax.nn.sigmoid(jnp.sum(x2 * t2, axis=1, keepdims=True))
            p2 = jnp.dot(oht, c2 * x2, preferred_element_type=jnp.float32)
            p1 = jnp.dot(oht, c1 * x1, preferred_element_type=jnp.float32)
            o_ref[:, 0:32] += p2
            o_ref[:, 32:96] += p1

    def _p1_idx(i):
        return jnp.where(i < _GRID, i, _GRID - 1)

    def _p2_idx(i):
        return jnp.where(i < _GRID, 0, i - _GRID)

    def _both_idx(i):
        return jnp.where(i < _GRID, i, i - _GRID)

    return pl.pallas_call(
        body, grid=(2 * _GRID,),
        in_specs=[
            pl.BlockSpec((1, _ROWBLK, 32), lambda i: (0, _p1_idx(i), 0)),
            pl.BlockSpec((1, _ROWBLK, 32), lambda i: (1, _p1_idx(i), 0)),
            pl.BlockSpec((_ROWBLK, 32), lambda i: (_p1_idx(i), 0)),
            pl.BlockSpec((_ROWBLK, 16), lambda i: (_p1_idx(i), 0)),
            pl.BlockSpec((8, 32), lambda i: (0, 0)),
            pl.BlockSpec((_ROWBLK, 16), lambda i: (_both_idx(i), 0)),
            pl.BlockSpec((_ROWBLK, 64), lambda i: (_p2_idx(i), 0)),
            pl.BlockSpec((_NG, 64), lambda i: (0, 0)),
            pl.BlockSpec((_NG, 16), lambda i: (0, 0)),
            pl.BlockSpec((32, 8), lambda i: (0, 0)),
            pl.BlockSpec((8, 32), lambda i: (0, 0)),
        ],
        out_specs=pl.BlockSpec((_NG, 96), lambda i: (0, 0)),
        out_shape=jax.ShapeDtypeStruct((_NG, 96), jnp.float32),
        scratch_shapes=[pltpu.VMEM((_N, 32), jnp.float32),
                        pltpu.VMEM((_NG, 32), jnp.float32)],
    )(acc2, acc2, g2, dv, b2, bt, x1, s1, cnt, w21, w22)


def _b2d(b):
    return jnp.broadcast_to(b[None, :], (8, b.shape[0]))


def kernel(edge_index, features, batch, W1, b1, W2, b2,
           att1_fc1, att1_fc2, att2_fc1, att2_fc2):
    src, dst = edge_index[0], edge_index[1]
    pad = _EPAD - _E
    spread = jnp.arange(pad, dtype=jnp.int32)
    src_p = jnp.concatenate([src, spread % _N]).reshape(_NW, _CHUNKS, _K)
    dst_p = jnp.concatenate([dst, _N + spread % (_NPAD - _N)]
                            ).reshape(_NW, _CHUNKS, _K)
    bt = jnp.broadcast_to(batch[:, None], (_N, 16))

    d_parts = _sc_degree(dst_p)
    h1 = _tc_mm1(features, W1)
    g1, dv = _tc_scale1(d_parts, h1)
    acc1 = _sc_scatter64(g1, src_p, dst_p)
    g2, x1, s1, cnt = _tc_layer2(acc1, g1, dv, _b2d(b1), W2,
                                 att1_fc1, att1_fc2, bt)
    acc2 = _sc_scatter32(g2, src_p, dst_p)
    return _tc_pool(acc2, g2, dv, _b2d(b2), bt, x1, s1, cnt,
                    att2_fc1, att2_fc2)


# speedup vs baseline: 36.3710x; 1.0338x over previous
import functools

import jax
import jax.numpy as jnp
from jax import lax
from jax.experimental import pallas as pl
from jax.experimental.pallas import tpu as pltpu
from jax.experimental.pallas import tpu_sc as plsc

_N = 10000
_E = 320000
_NG = 128

_NC = 2
_NS = 16
_NW = _NC * _NS
_K = 128
_CHUNKS = -(-_E // (_NW * _K))
_PER_W = _K * _CHUNKS
_EPAD = _NW * _PER_W
_NPAD = 10112
_RPT = _NPAD // _NS
_ROWBLK = 1000
_GRID = _N // _ROWBLK

_mesh = plsc.VectorSubcoreMesh(core_axis_name="c", subcore_axis_name="s")


_sc_params = pltpu.CompilerParams(use_tc_tiling_on_sc=False)
_NB = 4


@functools.partial(
    pl.kernel, mesh=_mesh, compiler_params=_sc_params,
    out_type=jax.ShapeDtypeStruct((_NC, _NPAD, 16), jnp.float32),
    scratch_types=[
        pltpu.VMEM((_CHUNKS, _K), jnp.int32),
        pltpu.VMEM((_K, 16), jnp.float32),
        pltpu.VMEM((_RPT, 16), jnp.float32),
        pltpu.VMEM_SHARED((_NPAD, 16), jnp.float32),
        pltpu.SemaphoreType.DMA,
    ])
def _sc_degree(dst_hbm, out_hbm, didx, ones_v, zbuf, acc, sem):
    c = lax.axis_index("c")
    s = lax.axis_index("s")
    wid = s * _NC + c

    def fill(i, _):
        ones_v[i, :] = jnp.ones((16,), jnp.float32)
        return 0
    lax.fori_loop(0, _K, fill, 0)

    def zfill(i, _):
        zbuf[i, :] = jnp.zeros((16,), jnp.float32)
        return 0
    lax.fori_loop(0, _RPT, zfill, 0)
    pltpu.sync_copy(zbuf, acc.at[pl.ds(s * _RPT, _RPT)])
    pltpu.sync_copy(dst_hbm.at[wid], didx)
    plsc.subcore_barrier()

    descs = [pltpu.make_async_copy(ones_v, acc.at[didx.at[i]], sem)
             for i in range(_CHUNKS)]
    for d in descs:
        d.start(add=True)
    for d in descs:
        d.wait()

    plsc.subcore_barrier()
    pltpu.sync_copy(acc.at[pl.ds(s * _RPT, _RPT)],
                    out_hbm.at[c, pl.ds(s * _RPT, _RPT)])


def _make_sc_scatter(width):
    @functools.partial(
        pl.kernel, mesh=_mesh, compiler_params=_sc_params,
        out_type=jax.ShapeDtypeStruct((_NC, _NPAD, width), jnp.float32),
        scratch_types=[
            pltpu.VMEM((_CHUNKS, _K), jnp.int32),
            pltpu.VMEM((_CHUNKS, _K), jnp.int32),
            pltpu.VMEM((_NB, _K, width), jnp.float32),
            pltpu.VMEM_SHARED((_NPAD, width), jnp.float32),
            pltpu.SemaphoreType.DMA((_NB,)),
            pltpu.SemaphoreType.DMA((_NB,)),
        ])
    def k(tab_hbm, src_hbm, dst_hbm, out_hbm, sidx, didx, rows, acc,
          gsem, ssem):
        c = lax.axis_index("c")
        s = lax.axis_index("s")
        wid = s * _NC + c

        def zfill(i, _):
            def zcol(j, _2):
                rows[0, i, pl.ds(j * 16, 16)] = jnp.zeros((16,), jnp.float32)
                return 0
            return lax.fori_loop(0, width // 16, zcol, 0)
        lax.fori_loop(0, _K, zfill, 0)
        nfull, rem = divmod(_RPT, _K)
        for t in range(nfull):
            pltpu.sync_copy(rows.at[0],
                            acc.at[pl.ds(s * _RPT + t * _K, _K)])
        if rem:
            pltpu.sync_copy(rows.at[0].at[pl.ds(0, rem)],
                            acc.at[pl.ds(s * _RPT + nfull * _K, rem)])
        pltpu.sync_copy(src_hbm.at[wid], sidx)
        pltpu.sync_copy(dst_hbm.at[wid], didx)
        plsc.subcore_barrier()

        def g_desc(j):
            return pltpu.make_async_copy(
                tab_hbm.at[sidx.at[j]], rows.at[j % _NB], gsem.at[j % _NB])

        def s_desc(i):
            return pltpu.make_async_copy(
                rows.at[i % _NB], acc.at[didx.at[i]], ssem.at[i % _NB])

        g_desc(0).start()
        g_desc(1).start()
        for i in range(_CHUNKS):
            j = i + 2
            if j < _CHUNKS:
                if j >= _NB:
                    s_desc(j - _NB).wait()
                g_desc(j).start()
            g_desc(i).wait()
            s_desc(i).start(add=True)
        for i in range(max(0, _CHUNKS - _NB), _CHUNKS):
            s_desc(i).wait()

        plsc.subcore_barrier()
        pltpu.sync_copy(acc.at[pl.ds(s * _RPT, _RPT)],
                        out_hbm.at[c, pl.ds(s * _RPT, _RPT)])
    return k


_sc_scatter64 = _make_sc_scatter(64)
_sc_scatter32 = _make_sc_scatter(32)


def _row_spec(width):
    return pl.BlockSpec((_ROWBLK, width), lambda i: (i, 0))


def _part_spec(core, width):
    return pl.BlockSpec((1, _ROWBLK, width), lambda i: (core, i, 0))


def _full_spec(r, cols):
    return pl.BlockSpec((r, cols), lambda i: (0, 0))


def _tc_mm1(x, w):
    def body(x_ref, w_ref, o_ref):
        o_ref[...] = jnp.dot(x_ref[...], w_ref[...],
                             preferred_element_type=jnp.float32)
    return pl.pallas_call(
        body, grid=(_GRID,),
        in_specs=[_row_spec(128), _full_spec(128, 64)],
        out_specs=_row_spec(64),
        out_shape=jax.ShapeDtypeStruct((_N, 64), jnp.float32),
    )(x, w)


def _tc_scale1(d_parts, h1):
    def body(d0_ref, d1_ref, h1_ref, g1_ref, dv_ref):
        deg = d0_ref[0][:, 0:1] + d1_ref[0][:, 0:1] + 1.0
        dv = lax.rsqrt(deg)
        g1_ref[...] = dv * h1_ref[...]
        dv_ref[...] = jnp.broadcast_to(dv, dv_ref.shape)
    return pl.pallas_call(
        body, grid=(_GRID,),
        in_specs=[_part_spec(0, 16), _part_spec(1, 16), _row_spec(64)],
        out_specs=[_row_spec(64), _row_spec(16)],
        out_shape=[jax.ShapeDtypeStruct((_N, 64), jnp.float32),
                   jax.ShapeDtypeStruct((_N, 16), jnp.float32)],
    )(d_parts, d_parts, h1)


def _p1_idx(i):
    return jnp.where(i < _GRID, i, _GRID - 1)


def _both_idx(i):
    return jnp.where(i < _GRID, i, i - _GRID)


def _tc_layer2a(acc1, g1, dv, b1, w2):
    def body(a0_ref, a1_ref, g1_ref, dv_ref, b1_ref, w2_ref,
             g2_ref, f1_ref):
        dv = dv_ref[...][:, 0:1]
        f1 = jnp.maximum(dv * (a0_ref[0] + a1_ref[0] + g1_ref[...])
                         + b1_ref[0:1, :], 0.0)
        f1_ref[...] = f1
        h2 = jnp.dot(f1, w2_ref[...], preferred_element_type=jnp.float32)
        g2_ref[...] = dv * h2
    return pl.pallas_call(
        body, grid=(_GRID,),
        in_specs=[_part_spec(0, 64), _part_spec(1, 64), _row_spec(64),
                  _row_spec(16), _full_spec(8, 64), _full_spec(64, 32)],
        out_specs=[_row_spec(32), _row_spec(64)],
        out_shape=[jax.ShapeDtypeStruct((_N, 32), jnp.float32),
                   jax.ShapeDtypeStruct((_N, 64), jnp.float32)],
    )(acc1, acc1, g1, dv, b1, w2)


def _tc_pool1(f1, w11, w12, bt):
    def body(f1_ref, w11_ref, w12_ref, bt_ref, p1_ref, cnt_ref, x1v, s1v):
        i = pl.program_id(0)

        @pl.when(i == 0)
        def _():
            s1v[...] = jnp.zeros_like(s1v)
            cnt_ref[...] = jnp.zeros_like(cnt_ref)
            p1_ref[...] = jnp.zeros_like(p1_ref)

        bk = bt_ref[...][:, 0:1]
        oht = (lax.broadcasted_iota(jnp.int32, (_NG, _ROWBLK), 0)
               == bk.reshape(1, _ROWBLK)).astype(jnp.float32)

        @pl.when(i < _GRID)
        def _():
            f1 = f1_ref[...]
            att1 = jnp.tanh(jnp.dot(
                jnp.maximum(jnp.dot(f1, w11_ref[...],
                                    preferred_element_type=jnp.float32), 0.0),
                w12_ref[...], preferred_element_type=jnp.float32))
            x1 = f1 + att1 * f1
            x1v[pl.ds(i * _ROWBLK, _ROWBLK), :] = x1
            s1v[...] += jnp.dot(oht, x1, preferred_element_type=jnp.float32)
            cnt = jnp.sum(oht, axis=1)[:, None]
            cnt_ref[...] += jnp.broadcast_to(cnt, cnt_ref.shape)

        @pl.when(i >= _GRID)
        def _():
            ib = i - _GRID
            oh = (bk == lax.broadcasted_iota(jnp.int32, (_ROWBLK, _NG), 1)
                  ).astype(jnp.float32)
            cntm = jnp.maximum(cnt_ref[...][:, 0:1], 1.0)
            tg1 = jnp.tanh(s1v[...] / cntm)
            x1 = x1v[pl.ds(ib * _ROWBLK, _ROWBLK), :]
            t1 = jnp.dot(oh, tg1, preferred_element_type=jnp.float32)
            c1 = jax.nn.sigmoid(jnp.sum(x1 * t1, axis=1, keepdims=True))
            p1_ref[...] += jnp.dot(oht, c1 * x1,
                                   preferred_element_type=jnp.float32)

    return pl.pallas_call(
        body, grid=(2 * _GRID,),
        in_specs=[
            pl.BlockSpec((_ROWBLK, 64), lambda i: (_p1_idx(i), 0)),
            pl.BlockSpec((64, 16), lambda i: (0, 0)),
            pl.BlockSpec((16, 64), lambda i: (0, 0)),
            pl.BlockSpec((_ROWBLK, 16), lambda i: (_both_idx(i), 0)),
        ],
        out_specs=[pl.BlockSpec((_NG, 64), lambda i: (0, 0)),
                   pl.BlockSpec((_NG, 16), lambda i: (0, 0))],
        out_shape=[jax.ShapeDtypeStruct((_NG, 64), jnp.float32),
                   jax.ShapeDtypeStruct((_NG, 16), jnp.float32)],
        scratch_shapes=[pltpu.VMEM((_N, 64), jnp.float32),
                        pltpu.VMEM((_NG, 64), jnp.float32)],
    )(f1, w11, w12, bt)


def _tc_pool2(acc2, g2, dv, b2, bt, p1, cnt, w21, w22):
    def body(a0_ref, a1_ref, g2_ref, dv_ref, b2_ref, bt_ref, p1_ref,
             cnt_ref, w21_ref, w22_ref, o_ref, x2v, s2v):
        i = pl.program_id(0)

        @pl.when(i == 0)
        def _():
            s2v[...] = jnp.zeros_like(s2v)
            o_ref[:, 0:32] = jnp.zeros_like(o_ref[:, 0:32])
            o_ref[:, 32:96] = p1_ref[...]

        bk = bt_ref[...][:, 0:1]
        oht = (lax.broadcasted_iota(jnp.int32, (_NG, _ROWBLK), 0)
               == bk.reshape(1, _ROWBLK)).astype(jnp.float32)

        @pl.when(i < _GRID)
        def _():
            dv = dv_ref[...][:, 0:1]
            f2 = jnp.maximum(dv * (a0_ref[0] + a1_ref[0] + g2_ref[...])
                             + b2_ref[0:1, :], 0.0)
            att2 = jnp.tanh(jnp.dot(
                jnp.maximum(jnp.dot(f2, w21_ref[...],
                                    preferred_element_type=jnp.float32), 0.0),
                w22_ref[...], preferred_element_type=jnp.float32))
            x2 = f2 + att2 * f2
            x2v[pl.ds(i * _ROWBLK, _ROWBLK), :] = x2
            s2v[...] += jnp.dot(oht, x2, preferred_element_type=jnp.float32)

        @pl.when(i >= _GRID)
        def _():
            ib = i - _GRID
            oh = (bk == lax.broadcasted_iota(jnp.int32, (_ROWBLK, _NG), 1)
                  ).astype(jnp.float32)
            cntm = jnp.maximum(cnt_ref[...][:, 0:1], 1.0)
            tg2 = jnp.tanh(s2v[...] / cntm)
            x2 = x2v[pl.ds(ib * _ROWBLK, _ROWBLK), :]
            t2 = jnp.dot(oh, tg2, preferred_element_type=jnp.float32)
            c2 = jax.nn.sigmoid(jnp.sum(x2 * t2, axis=1, keepdims=True))
            o_ref[:, 0:32] += jnp.dot(oht, c2 * x2,
                                      preferred_element_type=jnp.float32)

    return pl.pallas_call(
        body, grid=(2 * _GRID,),
        in_specs=[
            pl.BlockSpec((1, _ROWBLK, 32), lambda i: (0, _p1_idx(i), 0)),
            pl.BlockSpec((1, _ROWBLK, 32), lambda i: (1, _p1_idx(i), 0)),
            pl.BlockSpec((_ROWBLK, 32), lambda i: (_p1_idx(i), 0)),
            pl.BlockSpec((_ROWBLK, 16), lambda i: (_p1_idx(i), 0)),
            pl.BlockSpec((8, 32), lambda i: (0, 0)),
            pl.BlockSpec((_ROWBLK, 16), lambda i: (_both_idx(i), 0)),
            pl.BlockSpec((_NG, 64), lambda i: (0, 0)),
            pl.BlockSpec((_NG, 16), lambda i: (0, 0)),
            pl.BlockSpec((32, 8), lambda i: (0, 0)),
            pl.BlockSpec((8, 32), lambda i: (0, 0)),
        ],
        out_specs=pl.BlockSpec((_NG, 96), lambda i: (0, 0)),
        out_shape=jax.ShapeDtypeStruct((_NG, 96), jnp.float32),
        scratch_shapes=[pltpu.VMEM((_N, 32), jnp.float32),
                        pltpu.VMEM((_NG, 32), jnp.float32)],
    )(acc2, acc2, g2, dv, b2, bt, p1, cnt, w21, w22)


def _b2d(b):
    return jnp.broadcast_to(b[None, :], (8, b.shape[0]))


def kernel(edge_index, features, batch, W1, b1, W2, b2,
           att1_fc1, att1_fc2, att2_fc1, att2_fc2):
    src, dst = edge_index[0], edge_index[1]
    pad = _EPAD - _E
    spread = jnp.arange(pad, dtype=jnp.int32)
    src_p = jnp.concatenate([src, spread % _N]).reshape(_NW, _CHUNKS, _K)
    dst_p = jnp.concatenate([dst, _N + spread % (_NPAD - _N)]
                            ).reshape(_NW, _CHUNKS, _K)
    bt = jnp.broadcast_to(batch[:, None], (_N, 16))

    d_parts = _sc_degree(dst_p)
    h1 = _tc_mm1(features, W1)
    g1, dv = _tc_scale1(d_parts, h1)
    acc1 = _sc_scatter64(g1, src_p, dst_p)
    g2, f1 = _tc_layer2a(acc1, g1, dv, _b2d(b1), W2)
    acc2 = _sc_scatter32(g2, src_p, dst_p)
    p1, cnt = _tc_pool1(f1, att1_fc1, att1_fc2, bt)
    return _tc_pool2(acc2, g2, dv, _b2d(b2), bt, p1, cnt,
                     att2_fc1, att2_fc2)
